# Initial kernel scaffold; baseline (speedup 1.0000x reference)
import jax
import jax.numpy as jnp
from jax.experimental import pallas as pl


def kernel(x, W0, b0, W1, b1, W2, b2, W3, b3, edge_index):
    raise NotImplementedError("write your pallas kernel here")


# speedup vs baseline: 3.2384x; 3.2384x over previous
import functools
import math

import jax
import jax.numpy as jnp
from jax import lax
from jax.experimental import pallas as pl
from jax.experimental.pallas import tpu as pltpu
from jax.experimental.pallas import tpu_sc as plsc

N_NODES = 10000
D = 128
N_LAYERS = 4
ALPHA = 0.1
LAMBDA = 1.0

NC = 2
NS = 16
NW = NC * NS

CHUNK = 128
CH = 79
EPW = CH * CHUNK
N_PAD = 10240
RPS = N_PAD // NS
DEGW = 16


def _worker_id():
    return lax.axis_index("s") * NC + lax.axis_index("c")


def _zero_fill(ref, nrows, ncols):
    def row(i, _):
        def col(l, __):
            ref[i, pl.ds(l * 16, 16)] = jnp.zeros((16,), jnp.float32)
            return 0
        return lax.fori_loop(0, ncols // 16, col, 0)
    lax.fori_loop(0, nrows, row, 0)


def _sc_mesh():
    return plsc.VectorSubcoreMesh(core_axis_name="c", subcore_axis_name="s")


def _agg_body(h_hbm, srcp_hbm, dstp_hbm, out_hbm,
              agg_sh, idx_s, idx_d, rows, zbuf, gsem):
    cid = lax.axis_index("c")
    sid = lax.axis_index("s")
    wid = _worker_id()

    _zero_fill(zbuf, 64, D)
    for z in range(RPS // 64):
        pltpu.sync_copy(zbuf, agg_sh.at[pl.ds(sid * RPS + z * 64, 64)])
    pltpu.sync_copy(srcp_hbm.at[wid], idx_s)
    pltpu.sync_copy(dstp_hbm.at[wid], idx_d)
    plsc.subcore_barrier()

    def step(j, _):
        pltpu.async_copy(h_hbm.at[idx_s.at[j]], rows, gsem).wait()
        pltpu.sync_copy(rows, agg_sh.at[idx_d.at[j]], add=True)
        return 0
    lax.fori_loop(0, CH, step, 0)

    plsc.subcore_barrier()
    pltpu.sync_copy(agg_sh.at[pl.ds(sid * RPS, RPS)],
                    out_hbm.at[cid].at[pl.ds(sid * RPS, RPS)])


def _sc_aggregate(h, srcp, dstp):
    return pl.kernel(
        _agg_body,
        out_type=jax.ShapeDtypeStruct((NC, N_PAD, D), jnp.float32),
        mesh=_sc_mesh(),
        scratch_types=[
            pltpu.VMEM_SHARED((N_PAD, D), jnp.float32),
            pltpu.VMEM((CH, CHUNK), jnp.int32),
            pltpu.VMEM((CH, CHUNK), jnp.int32),
            pltpu.VMEM((CHUNK, D), jnp.float32),
            pltpu.VMEM((64, D), jnp.float32),
            pltpu.SemaphoreType.DMA,
        ],
    )(h, srcp, dstp)


ROWS_BLK = 1000
GRID = N_NODES // ROWS_BLK


def _prep_body(x_ref, degp_ref, norm_ref, hs_ref):
    deg = degp_ref[0, :, 0:1] + degp_ref[1, :, 0:1]
    nrm = lax.rsqrt(jnp.maximum(deg, 1.0))
    norm_ref[...] = nrm
    hs_ref[...] = x_ref[...] * nrm


def _tc_prep(x, degp):
    return pl.pallas_call(
        _prep_body,
        grid=(GRID,),
        in_specs=[
            pl.BlockSpec((ROWS_BLK, D), lambda i: (i, 0)),
            pl.BlockSpec((NC, ROWS_BLK, D), lambda i: (0, i, 0)),
        ],
        out_specs=[
            pl.BlockSpec((ROWS_BLK, 1), lambda i: (i, 0)),
            pl.BlockSpec((ROWS_BLK, D), lambda i: (i, 0)),
        ],
        out_shape=[
            jax.ShapeDtypeStruct((N_NODES, 1), jnp.float32),
            jax.ShapeDtypeStruct((N_NODES, D), jnp.float32),
        ],
    )(x, degp)


def _layer_body(beta, p_ref, x_ref, norm_ref, w_ref, b_ref, r_ref, rs_ref):
    agg = p_ref[0] + p_ref[1]
    nrm = norm_ref[...]
    t = (1.0 - ALPHA) * (agg * nrm) + ALPHA * x_ref[...]
    m = jnp.dot(t, w_ref[...], preferred_element_type=jnp.float32)
    r = (1.0 - beta) * t + beta * m + b_ref[...]
    r = jnp.maximum(r, 0.0)
    r_ref[...] = r
    rs_ref[...] = r * nrm


def _tc_layer(p, x, norm, W, b2d, beta):
    return pl.pallas_call(
        functools.partial(_layer_body, beta),
        grid=(GRID,),
        in_specs=[
            pl.BlockSpec((NC, ROWS_BLK, D), lambda i: (0, i, 0)),
            pl.BlockSpec((ROWS_BLK, D), lambda i: (i, 0)),
            pl.BlockSpec((ROWS_BLK, 1), lambda i: (i, 0)),
            pl.BlockSpec((D, D), lambda i: (0, 0)),
            pl.BlockSpec((1, D), lambda i: (0, 0)),
        ],
        out_specs=[
            pl.BlockSpec((ROWS_BLK, D), lambda i: (i, 0)),
            pl.BlockSpec((ROWS_BLK, D), lambda i: (i, 0)),
        ],
        out_shape=[
            jax.ShapeDtypeStruct((N_NODES, D), jnp.float32),
            jax.ShapeDtypeStruct((N_NODES, D), jnp.float32),
        ],
    )(p, x, norm, W, b2d)


def kernel(x, W0, b0, W1, b1, W2, b2, W3, b3, edge_index):
    Ws = (W0, W1, W2, W3)
    bs = (b0, b1, b2, b3)
    src = edge_index[0]
    dst = edge_index[1]
    e = src.shape[0]
    e_pad = NW * EPW
    srcp = jnp.concatenate(
        [src, jnp.zeros((e_pad - e,), jnp.int32)]).reshape(NW, CH, CHUNK)
    dstp = jnp.concatenate(
        [dst, jnp.full((e_pad - e,), N_PAD - 1, jnp.int32)]).reshape(NW, CH, CHUNK)

    degp = _sc_aggregate(jnp.ones((N_NODES, D), jnp.float32), srcp, dstp)
    norm, hs = _tc_prep(x, degp)

    r = x
    for i in range(N_LAYERS):
        beta = math.log(LAMBDA / (i + 1) + 1.0)
        p = _sc_aggregate(hs, srcp, dstp)
        r, hs = _tc_layer(p, x, norm, Ws[i], bs[i].reshape(1, D), beta)
    return r
